# initial kernel scaffold (unmeasured)
import jax
import jax.numpy as jnp
from jax import lax
from jax.experimental import pallas as pl
from jax.experimental.pallas import tpu as pltpu

N_DEV = 32


def kernel(x, w_mat, scale_x, scale_w):
    m_per, k = x.shape
    _, n = w_mat.shape
    m_total = N_DEV * m_per

    def body(x_ref, w_ref, sx_ref, sw_ref, out_ref,
             xfull_ref, w_bf16_ref, send_sems, recv_sems):
        my = lax.axis_index("i")
        left = lax.rem(my + (N_DEV - 1), N_DEV)
        right = lax.rem(my + 1, N_DEV)

        barrier_sem = pltpu.get_barrier_semaphore()
        for nbr in (left, right):
            pl.semaphore_signal(
                barrier_sem, inc=1,
                device_id=(nbr,), device_id_type=pl.DeviceIdType.MESH,
            )
        pl.semaphore_wait(barrier_sem, 2)

        s = sx_ref[0] * sw_ref[0]
        w_bf16_ref[...] = w_ref[...].astype(jnp.bfloat16)

        xfull_ref[0] = x_ref[...]
        out_ref[pl.ds(my * m_per, m_per), :] = (
            jnp.dot(x_ref[...].astype(jnp.bfloat16), w_bf16_ref[...],
                    preferred_element_type=jnp.float32) * s
        )

        for h in range(N_DEV - 1):
            rdma = pltpu.make_async_remote_copy(
                src_ref=xfull_ref.at[h],
                dst_ref=xfull_ref.at[h + 1],
                send_sem=send_sems.at[h],
                recv_sem=recv_sems.at[h],
                device_id=(right,),
                device_id_type=pl.DeviceIdType.MESH,
            )
            rdma.start()
            rdma.wait()
            origin = lax.rem(my - (h + 1) + N_DEV, N_DEV)
            out_ref[pl.ds(origin * m_per, m_per), :] = (
                jnp.dot(xfull_ref[h + 1].astype(jnp.bfloat16), w_bf16_ref[...],
                        preferred_element_type=jnp.float32) * s
            )

    return pl.pallas_call(
        body,
        out_shape=jax.ShapeDtypeStruct((m_total, n), jnp.float32),
        in_specs=[
            pl.BlockSpec(memory_space=pltpu.VMEM),
            pl.BlockSpec(memory_space=pltpu.VMEM),
            pl.BlockSpec(memory_space=pltpu.SMEM),
            pl.BlockSpec(memory_space=pltpu.SMEM),
        ],
        out_specs=pl.BlockSpec(memory_space=pltpu.VMEM),
        scratch_shapes=[
            pltpu.VMEM((N_DEV, m_per, k), x.dtype),
            pltpu.VMEM((k, n), jnp.bfloat16),
            pltpu.SemaphoreType.DMA((N_DEV - 1,)),
            pltpu.SemaphoreType.DMA((N_DEV - 1,)),
        ],
        compiler_params=pltpu.CompilerParams(collective_id=0),
    )(x, w_mat, scale_x, scale_w)


# baseline (device time: 253975 ns/iter reference)
import jax
import jax.numpy as jnp
from jax import lax
from jax.experimental import pallas as pl
from jax.experimental.pallas import tpu as pltpu

N_DEV = 32


def kernel(x, w_mat, scale_x, scale_w):
    m_per, k = x.shape
    _, n = w_mat.shape
    m_total = N_DEV * m_per

    def body(x_ref, w_ref, sx_ref, sw_ref, out_ref,
             xfull_ref, w_bf16_ref, send_sems, recv_sems):
        my = lax.axis_index("i")
        left = lax.rem(my + (N_DEV - 1), N_DEV)
        right = lax.rem(my + 1, N_DEV)

        barrier_sem = pltpu.get_barrier_semaphore()
        for nbr in (left, right):
            pl.semaphore_signal(
                barrier_sem, inc=1,
                device_id=(nbr,), device_id_type=pl.DeviceIdType.MESH,
            )
        pl.semaphore_wait(barrier_sem, 2)

        s = sx_ref[0] * sw_ref[0]
        w_bf16_ref[...] = w_ref[...].astype(jnp.bfloat16)

        xfull_ref[0] = x_ref[...].astype(jnp.float8_e4m3fn)
        out_ref[pl.ds(my * m_per, m_per), :] = (
            jnp.dot(x_ref[...].astype(jnp.bfloat16), w_bf16_ref[...],
                    preferred_element_type=jnp.float32) * s
        )

        for h in range(N_DEV - 1):
            rdma = pltpu.make_async_remote_copy(
                src_ref=xfull_ref.at[h],
                dst_ref=xfull_ref.at[h + 1],
                send_sem=send_sems.at[h],
                recv_sem=recv_sems.at[h],
                device_id=(right,),
                device_id_type=pl.DeviceIdType.MESH,
            )
            rdma.start()
            rdma.wait()
            origin = lax.rem(my - (h + 1) + N_DEV, N_DEV)
            out_ref[pl.ds(origin * m_per, m_per), :] = (
                jnp.dot(xfull_ref[h + 1].astype(jnp.bfloat16), w_bf16_ref[...],
                        preferred_element_type=jnp.float32) * s
            )

    return pl.pallas_call(
        body,
        out_shape=jax.ShapeDtypeStruct((m_total, n), jnp.float32),
        in_specs=[
            pl.BlockSpec(memory_space=pltpu.VMEM),
            pl.BlockSpec(memory_space=pltpu.VMEM),
            pl.BlockSpec(memory_space=pltpu.SMEM),
            pl.BlockSpec(memory_space=pltpu.SMEM),
        ],
        out_specs=pl.BlockSpec(memory_space=pltpu.VMEM),
        scratch_shapes=[
            pltpu.VMEM((N_DEV, m_per, k), jnp.float8_e4m3fn),
            pltpu.VMEM((k, n), jnp.bfloat16),
            pltpu.SemaphoreType.DMA((N_DEV - 1,)),
            pltpu.SemaphoreType.DMA((N_DEV - 1,)),
        ],
        compiler_params=pltpu.CompilerParams(collective_id=0),
    )(x, w_mat, scale_x, scale_w)


# device time: 189346 ns/iter; 1.3413x vs baseline; 1.3413x over previous
import jax
import jax.numpy as jnp
from jax import lax
from jax.experimental import pallas as pl
from jax.experimental.pallas import tpu as pltpu

N_DEV = 32
R_HOPS = N_DEV // 2
L_HOPS = N_DEV - 1 - R_HOPS


def kernel(x, w_mat, scale_x, scale_w):
    m_per, k = x.shape
    _, n = w_mat.shape
    m_total = N_DEV * m_per

    def body(x_ref, w_ref, sx_ref, sw_ref, out_ref,
             rbuf, lbuf, w_bf16_ref,
             r_send, r_recv, l_send, l_recv):
        my = lax.axis_index("i")
        left = lax.rem(my + (N_DEV - 1), N_DEV)
        right = lax.rem(my + 1, N_DEV)

        barrier_sem = pltpu.get_barrier_semaphore()
        for nbr in (left, right):
            pl.semaphore_signal(
                barrier_sem, inc=1,
                device_id=(nbr,), device_id_type=pl.DeviceIdType.MESH,
            )
        pl.semaphore_wait(barrier_sem, 2)

        s = sx_ref[0] * sw_ref[0]

        def rsend(h):
            r = pltpu.make_async_remote_copy(
                src_ref=rbuf.at[h], dst_ref=rbuf.at[h + 1],
                send_sem=r_send.at[h], recv_sem=r_recv.at[h],
                device_id=(right,), device_id_type=pl.DeviceIdType.MESH,
            )
            r.start()
            return r

        def lsend(h):
            r = pltpu.make_async_remote_copy(
                src_ref=lbuf.at[h], dst_ref=lbuf.at[h + 1],
                send_sem=l_send.at[h], recv_sem=l_recv.at[h],
                device_id=(left,), device_id_type=pl.DeviceIdType.MESH,
            )
            r.start()
            return r

        def gemm(src_ref, origin):
            out_ref[pl.ds(origin * m_per, m_per), :] = (
                jnp.dot(src_ref[...].astype(jnp.bfloat16), w_bf16_ref[...],
                        preferred_element_type=jnp.float32) * s
            )

        x8 = x_ref[...].astype(jnp.float8_e4m3fn)
        rbuf[0] = x8
        lbuf[0] = x8
        rdma_r = rsend(0)
        rdma_l = lsend(0)

        w_bf16_ref[...] = w_ref[...].astype(jnp.bfloat16)
        gemm(x_ref, my)

        for h in range(R_HOPS):
            rdma_r.wait_recv()
            if h + 1 < R_HOPS:
                next_r = rsend(h + 1)
            if h < L_HOPS:
                rdma_l.wait_recv()
                if h + 1 < L_HOPS:
                    next_l = lsend(h + 1)
            gemm(rbuf.at[h + 1], lax.rem(my - (h + 1) + N_DEV, N_DEV))
            if h < L_HOPS:
                gemm(lbuf.at[h + 1], lax.rem(my + h + 1, N_DEV))
                rdma_l.wait_send()
                if h + 1 < L_HOPS:
                    rdma_l = next_l
            rdma_r.wait_send()
            if h + 1 < R_HOPS:
                rdma_r = next_r

    return pl.pallas_call(
        body,
        out_shape=jax.ShapeDtypeStruct((m_total, n), jnp.float32),
        in_specs=[
            pl.BlockSpec(memory_space=pltpu.VMEM),
            pl.BlockSpec(memory_space=pltpu.VMEM),
            pl.BlockSpec(memory_space=pltpu.SMEM),
            pl.BlockSpec(memory_space=pltpu.SMEM),
        ],
        out_specs=pl.BlockSpec(memory_space=pltpu.VMEM),
        scratch_shapes=[
            pltpu.VMEM((R_HOPS + 1, m_per, k), jnp.float8_e4m3fn),
            pltpu.VMEM((L_HOPS + 1, m_per, k), jnp.float8_e4m3fn),
            pltpu.VMEM((k, n), jnp.bfloat16),
            pltpu.SemaphoreType.DMA((R_HOPS,)),
            pltpu.SemaphoreType.DMA((R_HOPS,)),
            pltpu.SemaphoreType.DMA((L_HOPS,)),
            pltpu.SemaphoreType.DMA((L_HOPS,)),
        ],
        compiler_params=pltpu.CompilerParams(collective_id=0),
    )(x, w_mat, scale_x, scale_w)


# device time: 128893 ns/iter; 1.9704x vs baseline; 1.4690x over previous
import jax
import jax.numpy as jnp
from jax import lax
from jax.experimental import pallas as pl
from jax.experimental.pallas import tpu as pltpu

N_DEV = 32
R_HOPS = N_DEV // 2
L_HOPS = N_DEV - 1 - R_HOPS

def _mesh_index(x, y, z):
    return z * 8 + y * 2 + (x if y % 2 == 0 else 1 - x)

_RING_COORDS = (
    [(0, y, z) for y in range(4) for z in (range(4) if y % 2 == 0 else range(3, -1, -1))]
    + [(1, y, z) for y in range(3, -1, -1) for z in (range(4) if y % 2 == 1 else range(3, -1, -1))]
)
assert len(set(_RING_COORDS)) == N_DEV
for _p in range(N_DEV):
    _a, _b = _RING_COORDS[_p], _RING_COORDS[(_p + 1) % N_DEV]
    assert sum(abs(_a[i] - _b[i]) for i in range(3)) == 1, (_p, _a, _b)

_MESH_OF_RING = [_mesh_index(*c) for c in _RING_COORDS]
_RING_OF_MESH = [0] * N_DEV
for _p, _m in enumerate(_MESH_OF_RING):
    _RING_OF_MESH[_m] = _p


def kernel(x, w_mat, scale_x, scale_w):
    m_per, k = x.shape
    _, n = w_mat.shape
    m_total = N_DEV * m_per

    mesh_of_ring = jnp.asarray(_MESH_OF_RING, jnp.int32)
    ring_of_mesh = jnp.asarray(_RING_OF_MESH, jnp.int32)
    my_mesh = lax.axis_index("i")
    r = ring_of_mesh[my_mesh]
    nbrs = mesh_of_ring[jnp.stack([(r + 1) % N_DEV, (r - 1) % N_DEV])]
    r_orig = mesh_of_ring[(r - 1 - jnp.arange(R_HOPS)) % N_DEV]
    l_orig = mesh_of_ring[(r + 1 + jnp.arange(L_HOPS)) % N_DEV]

    def body(x_ref, w_ref, sx_ref, sw_ref, nbrs_ref, rorig_ref, lorig_ref,
             out_ref, rbuf, lbuf, w_bf16_ref,
             r_send, r_recv, l_send, l_recv):
        right = nbrs_ref[0]
        left = nbrs_ref[1]

        barrier_sem = pltpu.get_barrier_semaphore()
        for nbr in (left, right):
            pl.semaphore_signal(
                barrier_sem, inc=1,
                device_id=(nbr,), device_id_type=pl.DeviceIdType.MESH,
            )
        pl.semaphore_wait(barrier_sem, 2)

        s = sx_ref[0] * sw_ref[0]

        def rsend(h):
            c = pltpu.make_async_remote_copy(
                src_ref=rbuf.at[h], dst_ref=rbuf.at[h + 1],
                send_sem=r_send.at[h], recv_sem=r_recv.at[h],
                device_id=(right,), device_id_type=pl.DeviceIdType.MESH,
            )
            c.start()
            return c

        def lsend(h):
            c = pltpu.make_async_remote_copy(
                src_ref=lbuf.at[h], dst_ref=lbuf.at[h + 1],
                send_sem=l_send.at[h], recv_sem=l_recv.at[h],
                device_id=(left,), device_id_type=pl.DeviceIdType.MESH,
            )
            c.start()
            return c

        def gemm(src_ref, origin):
            out_ref[pl.ds(origin * m_per, m_per), :] = (
                jnp.dot(src_ref[...].astype(jnp.bfloat16), w_bf16_ref[...],
                        preferred_element_type=jnp.float32) * s
            )

        x8 = x_ref[...].astype(jnp.float8_e4m3fn)
        rbuf[0] = x8
        lbuf[0] = x8
        rdma_r = rsend(0)
        rdma_l = lsend(0)

        w_bf16_ref[...] = w_ref[...].astype(jnp.bfloat16)
        gemm(x_ref, lax.axis_index("i"))

        for h in range(R_HOPS):
            rdma_r.wait_recv()
            if h + 1 < R_HOPS:
                next_r = rsend(h + 1)
            if h < L_HOPS:
                rdma_l.wait_recv()
                if h + 1 < L_HOPS:
                    next_l = lsend(h + 1)
            gemm(rbuf.at[h + 1], rorig_ref[h])
            if h < L_HOPS:
                gemm(lbuf.at[h + 1], lorig_ref[h])
                rdma_l.wait_send()
                if h + 1 < L_HOPS:
                    rdma_l = next_l
            rdma_r.wait_send()
            if h + 1 < R_HOPS:
                rdma_r = next_r

    return pl.pallas_call(
        body,
        out_shape=jax.ShapeDtypeStruct((m_total, n), jnp.float32),
        in_specs=[
            pl.BlockSpec(memory_space=pltpu.VMEM),
            pl.BlockSpec(memory_space=pltpu.VMEM),
            pl.BlockSpec(memory_space=pltpu.SMEM),
            pl.BlockSpec(memory_space=pltpu.SMEM),
            pl.BlockSpec(memory_space=pltpu.SMEM),
            pl.BlockSpec(memory_space=pltpu.SMEM),
            pl.BlockSpec(memory_space=pltpu.SMEM),
        ],
        out_specs=pl.BlockSpec(memory_space=pltpu.VMEM),
        scratch_shapes=[
            pltpu.VMEM((R_HOPS + 1, m_per, k), jnp.float8_e4m3fn),
            pltpu.VMEM((L_HOPS + 1, m_per, k), jnp.float8_e4m3fn),
            pltpu.VMEM((k, n), jnp.bfloat16),
            pltpu.SemaphoreType.DMA((R_HOPS,)),
            pltpu.SemaphoreType.DMA((R_HOPS,)),
            pltpu.SemaphoreType.DMA((L_HOPS,)),
            pltpu.SemaphoreType.DMA((L_HOPS,)),
        ],
        compiler_params=pltpu.CompilerParams(collective_id=0),
    )(x, w_mat, scale_x, scale_w, nbrs, r_orig, l_orig)


# device time: 102773 ns/iter; 2.4712x vs baseline; 1.2542x over previous
import jax
import jax.numpy as jnp
from jax import lax
from jax.experimental import pallas as pl
from jax.experimental.pallas import tpu as pltpu

N_DEV = 32
R_HOPS = N_DEV // 2
L_HOPS = N_DEV - 1 - R_HOPS
SPLIT = 2

def _mesh_index(x, y, z):
    return z * 8 + y * 2 + (x if y % 2 == 0 else 1 - x)

_RING_COORDS = (
    [(0, y, z) for y in range(4) for z in (range(4) if y % 2 == 0 else range(3, -1, -1))]
    + [(1, y, z) for y in range(3, -1, -1) for z in (range(4) if y % 2 == 1 else range(3, -1, -1))]
)
assert len(set(_RING_COORDS)) == N_DEV
for _p in range(N_DEV):
    _a, _b = _RING_COORDS[_p], _RING_COORDS[(_p + 1) % N_DEV]
    assert sum(abs(_a[i] - _b[i]) for i in range(3)) == 1, (_p, _a, _b)

_MESH_OF_RING = [_mesh_index(*c) for c in _RING_COORDS]
_RING_OF_MESH = [0] * N_DEV
for _p, _m in enumerate(_MESH_OF_RING):
    _RING_OF_MESH[_m] = _p


def kernel(x, w_mat, scale_x, scale_w):
    m_per, k = x.shape
    _, n = w_mat.shape
    m_total = N_DEV * m_per

    mesh_of_ring = jnp.asarray(_MESH_OF_RING, jnp.int32)
    ring_of_mesh = jnp.asarray(_RING_OF_MESH, jnp.int32)
    my_mesh = lax.axis_index("i")
    r = ring_of_mesh[my_mesh]
    nbrs = mesh_of_ring[jnp.stack([(r + 1) % N_DEV, (r - 1) % N_DEV])]
    r_orig = mesh_of_ring[(r - 1 - jnp.arange(R_HOPS)) % N_DEV]
    l_orig = mesh_of_ring[(r + 1 + jnp.arange(L_HOPS)) % N_DEV]

    def body(x_ref, w_ref, sx_ref, sw_ref, nbrs_ref, rorig_ref, lorig_ref,
             out_ref, rbuf, lbuf, w_bf16_ref,
             r_send, r_recv, l_send, l_recv):
        right = nbrs_ref[0]
        left = nbrs_ref[1]

        barrier_sem = pltpu.get_barrier_semaphore()
        for nbr in (left, right):
            pl.semaphore_signal(
                barrier_sem, inc=1,
                device_id=(nbr,), device_id_type=pl.DeviceIdType.MESH,
            )
        pl.semaphore_wait(barrier_sem, 2)

        s = sx_ref[0] * sw_ref[0]
        rows = m_per // SPLIT

        def rsend(h, j):
            c = pltpu.make_async_remote_copy(
                src_ref=rbuf.at[h, pl.ds(j * rows, rows)],
                dst_ref=rbuf.at[h + 1, pl.ds(j * rows, rows)],
                send_sem=r_send.at[h * SPLIT + j],
                recv_sem=r_recv.at[h * SPLIT + j],
                device_id=(right,), device_id_type=pl.DeviceIdType.MESH,
            )
            c.start()
            return c

        def lsend(h, j):
            c = pltpu.make_async_remote_copy(
                src_ref=lbuf.at[h, pl.ds(j * rows, rows)],
                dst_ref=lbuf.at[h + 1, pl.ds(j * rows, rows)],
                send_sem=l_send.at[h * SPLIT + j],
                recv_sem=l_recv.at[h * SPLIT + j],
                device_id=(left,), device_id_type=pl.DeviceIdType.MESH,
            )
            c.start()
            return c

        def gemm(src_ref, origin):
            out_ref[pl.ds(origin * m_per, m_per), :] = (
                jnp.dot(src_ref[...].astype(jnp.bfloat16), w_bf16_ref[...],
                        preferred_element_type=jnp.float32) * s
            )

        x8 = x_ref[...].astype(jnp.float8_e4m3fn)
        rbuf[0] = x8
        lbuf[0] = x8
        rdmas_r = [rsend(0, j) for j in range(SPLIT)]
        rdmas_l = [lsend(0, j) for j in range(SPLIT)]

        w_bf16_ref[...] = w_ref[...].astype(jnp.bfloat16)
        gemm(x_ref, lax.axis_index("i"))

        for h in range(R_HOPS):
            next_r = [None] * SPLIT
            next_l = [None] * SPLIT
            for j in range(SPLIT):
                rdmas_r[j].wait_recv()
                if h + 1 < R_HOPS:
                    next_r[j] = rsend(h + 1, j)
                if h < L_HOPS:
                    rdmas_l[j].wait_recv()
                    if h + 1 < L_HOPS:
                        next_l[j] = lsend(h + 1, j)
            gemm(rbuf.at[h + 1], rorig_ref[h])
            if h < L_HOPS:
                gemm(lbuf.at[h + 1], lorig_ref[h])
                for j in range(SPLIT):
                    rdmas_l[j].wait_send()
                rdmas_l = next_l
            for j in range(SPLIT):
                rdmas_r[j].wait_send()
            rdmas_r = next_r

    return pl.pallas_call(
        body,
        out_shape=jax.ShapeDtypeStruct((m_total, n), jnp.float32),
        in_specs=[
            pl.BlockSpec(memory_space=pltpu.VMEM),
            pl.BlockSpec(memory_space=pltpu.VMEM),
            pl.BlockSpec(memory_space=pltpu.SMEM),
            pl.BlockSpec(memory_space=pltpu.SMEM),
            pl.BlockSpec(memory_space=pltpu.SMEM),
            pl.BlockSpec(memory_space=pltpu.SMEM),
            pl.BlockSpec(memory_space=pltpu.SMEM),
        ],
        out_specs=pl.BlockSpec(memory_space=pltpu.VMEM),
        scratch_shapes=[
            pltpu.VMEM((R_HOPS + 1, m_per, k), jnp.float8_e4m3fn),
            pltpu.VMEM((L_HOPS + 1, m_per, k), jnp.float8_e4m3fn),
            pltpu.VMEM((k, n), jnp.bfloat16),
            pltpu.SemaphoreType.DMA((R_HOPS * SPLIT,)),
            pltpu.SemaphoreType.DMA((R_HOPS * SPLIT,)),
            pltpu.SemaphoreType.DMA((L_HOPS * SPLIT,)),
            pltpu.SemaphoreType.DMA((L_HOPS * SPLIT,)),
        ],
        compiler_params=pltpu.CompilerParams(collective_id=0),
    )(x, w_mat, scale_x, scale_w, nbrs, r_orig, l_orig)
